# baseline (speedup 1.0000x reference)
import functools

import jax
import jax.numpy as jnp
from jax import lax
from jax.experimental import pallas as pl
from jax.experimental.pallas import tpu as pltpu
from jax.experimental.pallas import tpu_sc as plsc

Q_BLK = 256
S_BLK = 2048
G_BLK = 512
D_PAD = 16
_LANES = 128

_SC_CORES = 2
_SC_SUBCORES = 16
_SC_WORKERS = _SC_CORES * _SC_SUBCORES


def _mlp_body(x_ref, w1_ref, b1_ref, w2_ref, b2_ref, w3_ref, b3_ref, out_ref):
    h = jnp.maximum(jnp.dot(x_ref[...], w1_ref[...]) + b1_ref[...], 0.0)
    h = jnp.maximum(jnp.dot(h, w2_ref[...]) + b2_ref[...], 0.0)
    out_ref[...] = jnp.dot(h, w3_ref[...]) + b3_ref[...]


def _snorm_body(s_ref, out_ref):
    st = s_ref[...]
    out_ref[...] = st / jnp.maximum(
        jnp.sqrt(jnp.sum(st * st, axis=0, keepdims=True)), 1e-8)


def _sims_body(ns_tiles, q_ref, s_ref, out_ref, qn_ref, vmax_ref, vidx_ref):
    j = pl.program_id(1)

    @pl.when(j == 0)
    def _init():
        qt = q_ref[...]
        qn_ref[...] = qt / jnp.maximum(
            jnp.sqrt(jnp.sum(qt * qt, axis=1, keepdims=True)), 1e-8)
        vmax_ref[...] = jnp.full((Q_BLK, _LANES), -jnp.inf, jnp.float32)
        vidx_ref[...] = jnp.zeros((Q_BLK, _LANES), jnp.int32)

    lane = lax.broadcasted_iota(jnp.int32, (Q_BLK, _LANES), 1)
    nparts = G_BLK // _LANES
    for g in range(S_BLK // G_BLK):
        sims = jnp.dot(qn_ref[...], s_ref[:, g * G_BLK:(g + 1) * G_BLK])
        vals = [sims[:, k * _LANES:(k + 1) * _LANES] for k in range(nparts)]
        idxs = [jnp.full((Q_BLK, _LANES), k * _LANES, jnp.int32)
                for k in range(nparts)]
        while len(vals) > 1:
            nv, ni = [], []
            for a in range(0, len(vals), 2):
                m = vals[a + 1] > vals[a]
                nv.append(jnp.where(m, vals[a + 1], vals[a]))
                ni.append(jnp.where(m, idxs[a + 1], idxs[a]))
            vals, idxs = nv, ni
        tile_idx = idxs[0] + (lane + (j * S_BLK + g * G_BLK))
        m = vals[0] > vmax_ref[...]
        vmax_ref[...] = jnp.where(m, vals[0], vmax_ref[...])
        vidx_ref[...] = jnp.where(m, tile_idx, vidx_ref[...])

    @pl.when(j == ns_tiles - 1)
    def _finish():
        vm = vmax_ref[...]
        rowmax = jnp.max(vm, axis=1, keepdims=True)
        cand = jnp.where(vm == rowmax, vidx_ref[...], jnp.int32(2**31 - 1))
        out_ref[0, 0, :] = jnp.min(cand, axis=1)


def _gather_body(b_per_w, table_ref, idx_ref, out_ref, idx_v, rows_v, sem):
    wid = lax.axis_index("s") * _SC_CORES + lax.axis_index("c")
    base = wid * b_per_w
    pltpu.sync_copy(idx_ref.at[pl.ds(base, b_per_w)], idx_v)
    pltpu.async_copy(table_ref.at[idx_v], rows_v, sem).wait()
    pltpu.sync_copy(rows_v, out_ref.at[pl.ds(base, b_per_w)])


def kernel(x, songs, W1, b1, W2, b2, W3, b3):
    b = x.shape[0]
    n_songs, d = songs.shape
    xf = x.reshape(b, -1)

    produced = pl.pallas_call(
        _mlp_body,
        out_shape=jax.ShapeDtypeStruct((b, W3.shape[1]), jnp.float32),
    )(xf, W1, b1.reshape(1, -1), W2, b2.reshape(1, -1), W3, b3.reshape(1, -1))

    q = produced.reshape(-1, d)
    nq = q.shape[0]
    k_out = produced.shape[1] // d
    q_pad = jnp.pad(q, ((0, 0), (0, D_PAD - d)))

    ns_tiles = -(-n_songs // S_BLK)
    s_cols = ns_tiles * S_BLK
    table = jnp.pad(songs, ((0, 0), (0, D_PAD - d)))
    songs_t = table.T
    songs_t = jnp.concatenate(
        [songs_t, jnp.broadcast_to(songs_t[:, :1], (D_PAD, s_cols - n_songs))],
        axis=1)

    songs_tn = pl.pallas_call(
        _snorm_body,
        grid=(ns_tiles,),
        in_specs=[pl.BlockSpec((D_PAD, S_BLK), lambda j: (0, j))],
        out_specs=pl.BlockSpec((D_PAD, S_BLK), lambda j: (0, j)),
        out_shape=jax.ShapeDtypeStruct((D_PAD, s_cols), jnp.float32),
    )(songs_t)

    nq_tiles = nq // Q_BLK
    idx3 = pl.pallas_call(
        functools.partial(_sims_body, ns_tiles),
        grid=(nq_tiles, ns_tiles),
        in_specs=[
            pl.BlockSpec((Q_BLK, D_PAD), lambda i, j: (i, 0)),
            pl.BlockSpec((D_PAD, S_BLK), lambda i, j: (0, j)),
        ],
        out_specs=pl.BlockSpec((1, 1, Q_BLK), lambda i, j: (i, 0, 0)),
        out_shape=jax.ShapeDtypeStruct((nq_tiles, 1, Q_BLK), jnp.int32),
        scratch_shapes=[
            pltpu.VMEM((Q_BLK, D_PAD), jnp.float32),
            pltpu.VMEM((Q_BLK, _LANES), jnp.float32),
            pltpu.VMEM((Q_BLK, _LANES), jnp.int32),
        ],
        compiler_params=pltpu.CompilerParams(
            dimension_semantics=("parallel", "arbitrary"),
        ),
    )(q_pad, songs_tn)
    idx = idx3.reshape(nq)

    b_per_w = nq // _SC_WORKERS
    mesh = plsc.VectorSubcoreMesh(core_axis_name="c", subcore_axis_name="s")
    gathered = pl.kernel(
        functools.partial(_gather_body, b_per_w),
        mesh=mesh,
        out_type=jax.ShapeDtypeStruct((nq, D_PAD), jnp.float32),
        scratch_types=[
            pltpu.VMEM((b_per_w,), jnp.int32),
            pltpu.VMEM((b_per_w, D_PAD), jnp.float32),
            pltpu.SemaphoreType.DMA,
        ],
        compiler_params=pltpu.CompilerParams(use_tc_tiling_on_sc=False),
    )(table, idx)

    return gathered[:, :d].reshape(b, k_out, d)


# speedup vs baseline: 1.0027x; 1.0027x over previous
import functools

import jax
import jax.numpy as jnp
from jax import lax
from jax.experimental import pallas as pl
from jax.experimental.pallas import tpu as pltpu
from jax.experimental.pallas import tpu_sc as plsc

Q_BLK = 256
S_BLK = 2048
G_BLK = 128
D_PAD = 16
_LANES = 128

_SC_CORES = 2
_SC_SUBCORES = 16
_SC_WORKERS = _SC_CORES * _SC_SUBCORES


def _mlp_body(x_ref, w1_ref, b1_ref, w2_ref, b2_ref, w3_ref, b3_ref, out_ref):
    h = jnp.maximum(jnp.dot(x_ref[...], w1_ref[...]) + b1_ref[...], 0.0)
    h = jnp.maximum(jnp.dot(h, w2_ref[...]) + b2_ref[...], 0.0)
    out_ref[...] = jnp.dot(h, w3_ref[...]) + b3_ref[...]


def _snorm_body(s_ref, out_ref):
    st = s_ref[...]
    out_ref[...] = st / jnp.maximum(
        jnp.sqrt(jnp.sum(st * st, axis=0, keepdims=True)), 1e-8)


def _sims_body(ns_tiles, q_ref, s_ref, out_ref, qn_ref, vmax_ref, vidx_ref):
    j = pl.program_id(1)

    @pl.when(j == 0)
    def _init():
        qt = q_ref[...]
        qn_ref[...] = qt / jnp.maximum(
            jnp.sqrt(jnp.sum(qt * qt, axis=1, keepdims=True)), 1e-8)
        vmax_ref[...] = jnp.full((Q_BLK, _LANES), -jnp.inf, jnp.float32)
        vidx_ref[...] = jnp.zeros((Q_BLK, _LANES), jnp.int32)

    lane = lax.broadcasted_iota(jnp.int32, (Q_BLK, _LANES), 1)
    nparts = G_BLK // _LANES
    for g in range(S_BLK // G_BLK):
        sims = jnp.dot(qn_ref[...], s_ref[:, g * G_BLK:(g + 1) * G_BLK])
        vals = [sims[:, k * _LANES:(k + 1) * _LANES] for k in range(nparts)]
        idxs = [jnp.full((Q_BLK, _LANES), k * _LANES, jnp.int32)
                for k in range(nparts)]
        while len(vals) > 1:
            nv, ni = [], []
            for a in range(0, len(vals), 2):
                m = vals[a + 1] > vals[a]
                nv.append(jnp.where(m, vals[a + 1], vals[a]))
                ni.append(jnp.where(m, idxs[a + 1], idxs[a]))
            vals, idxs = nv, ni
        tile_idx = idxs[0] + (lane + (j * S_BLK + g * G_BLK))
        m = vals[0] > vmax_ref[...]
        vmax_ref[...] = jnp.where(m, vals[0], vmax_ref[...])
        vidx_ref[...] = jnp.where(m, tile_idx, vidx_ref[...])

    @pl.when(j == ns_tiles - 1)
    def _finish():
        vm = vmax_ref[...]
        rowmax = jnp.max(vm, axis=1, keepdims=True)
        cand = jnp.where(vm == rowmax, vidx_ref[...], jnp.int32(2**31 - 1))
        out_ref[0, 0, :] = jnp.min(cand, axis=1)


def _gather_body(b_per_w, table_ref, idx_ref, out_ref, idx_v, rows_v, sem):
    wid = lax.axis_index("s") * _SC_CORES + lax.axis_index("c")
    base = wid * b_per_w
    pltpu.sync_copy(idx_ref.at[pl.ds(base, b_per_w)], idx_v)
    pltpu.async_copy(table_ref.at[idx_v], rows_v, sem).wait()
    pltpu.sync_copy(rows_v, out_ref.at[pl.ds(base, b_per_w)])


def kernel(x, songs, W1, b1, W2, b2, W3, b3):
    b = x.shape[0]
    n_songs, d = songs.shape
    xf = x.reshape(b, -1)

    produced = pl.pallas_call(
        _mlp_body,
        out_shape=jax.ShapeDtypeStruct((b, W3.shape[1]), jnp.float32),
    )(xf, W1, b1.reshape(1, -1), W2, b2.reshape(1, -1), W3, b3.reshape(1, -1))

    q = produced.reshape(-1, d)
    nq = q.shape[0]
    k_out = produced.shape[1] // d
    q_pad = jnp.pad(q, ((0, 0), (0, D_PAD - d)))

    ns_tiles = -(-n_songs // S_BLK)
    s_cols = ns_tiles * S_BLK
    table = jnp.pad(songs, ((0, 0), (0, D_PAD - d)))
    songs_t = table.T
    songs_t = jnp.concatenate(
        [songs_t, jnp.broadcast_to(songs_t[:, :1], (D_PAD, s_cols - n_songs))],
        axis=1)

    songs_tn = pl.pallas_call(
        _snorm_body,
        grid=(ns_tiles,),
        in_specs=[pl.BlockSpec((D_PAD, S_BLK), lambda j: (0, j))],
        out_specs=pl.BlockSpec((D_PAD, S_BLK), lambda j: (0, j)),
        out_shape=jax.ShapeDtypeStruct((D_PAD, s_cols), jnp.float32),
    )(songs_t)

    nq_tiles = nq // Q_BLK
    idx3 = pl.pallas_call(
        functools.partial(_sims_body, ns_tiles),
        grid=(nq_tiles, ns_tiles),
        in_specs=[
            pl.BlockSpec((Q_BLK, D_PAD), lambda i, j: (i, 0)),
            pl.BlockSpec((D_PAD, S_BLK), lambda i, j: (0, j)),
        ],
        out_specs=pl.BlockSpec((1, 1, Q_BLK), lambda i, j: (i, 0, 0)),
        out_shape=jax.ShapeDtypeStruct((nq_tiles, 1, Q_BLK), jnp.int32),
        scratch_shapes=[
            pltpu.VMEM((Q_BLK, D_PAD), jnp.float32),
            pltpu.VMEM((Q_BLK, _LANES), jnp.float32),
            pltpu.VMEM((Q_BLK, _LANES), jnp.int32),
        ],
        compiler_params=pltpu.CompilerParams(
            dimension_semantics=("parallel", "arbitrary"),
        ),
    )(q_pad, songs_tn)
    idx = idx3.reshape(nq)

    b_per_w = nq // _SC_WORKERS
    mesh = plsc.VectorSubcoreMesh(core_axis_name="c", subcore_axis_name="s")
    gathered = pl.kernel(
        functools.partial(_gather_body, b_per_w),
        mesh=mesh,
        out_type=jax.ShapeDtypeStruct((nq, D_PAD), jnp.float32),
        scratch_types=[
            pltpu.VMEM((b_per_w,), jnp.int32),
            pltpu.VMEM((b_per_w, D_PAD), jnp.float32),
            pltpu.SemaphoreType.DMA,
        ],
        compiler_params=pltpu.CompilerParams(use_tc_tiling_on_sc=False),
    )(table, idx)

    return gathered[:, :d].reshape(b, k_out, d)


# speedup vs baseline: 1.4902x; 1.4861x over previous
import functools

import jax
import jax.numpy as jnp
from jax import lax
from jax.experimental import pallas as pl
from jax.experimental.pallas import tpu as pltpu
from jax.experimental.pallas import tpu_sc as plsc

Q_BLK = 256
S_BLK = 4096
G_BLK = 128
D_PAD = 16
_LANES = 128

_SC_CORES = 2
_SC_SUBCORES = 16
_SC_WORKERS = _SC_CORES * _SC_SUBCORES


def _mlp_body(x_ref, w1_ref, b1_ref, w2_ref, b2_ref, w3_ref, b3_ref, out_ref):
    h = jnp.maximum(jnp.dot(x_ref[...], w1_ref[...]) + b1_ref[...], 0.0)
    h = jnp.maximum(jnp.dot(h, w2_ref[...]) + b2_ref[...], 0.0)
    out_ref[...] = jnp.dot(h, w3_ref[...]) + b3_ref[...]


def _snorm_body(s_ref, out_ref):
    st = s_ref[...]
    out_ref[...] = st / jnp.maximum(
        jnp.sqrt(jnp.sum(st * st, axis=0, keepdims=True)), 1e-8)


def _sims_body(ns_tiles, q_ref, s_ref, out_ref, qn_ref, vmax_ref, vidx_ref):
    j = pl.program_id(1)

    @pl.when(j == 0)
    def _init():
        qt = q_ref[...]
        qn_ref[...] = qt / jnp.maximum(
            jnp.sqrt(jnp.sum(qt * qt, axis=1, keepdims=True)), 1e-8)
        vmax_ref[...] = jnp.full((Q_BLK, _LANES), -jnp.inf, jnp.float32)
        vidx_ref[...] = jnp.zeros((Q_BLK, _LANES), jnp.int32)

    lane = lax.broadcasted_iota(jnp.int32, (Q_BLK, _LANES), 1)
    nparts = G_BLK // _LANES
    for g in range(S_BLK // G_BLK):
        sims = jnp.dot(qn_ref[...], s_ref[:, g * G_BLK:(g + 1) * G_BLK])
        vals = [sims[:, k * _LANES:(k + 1) * _LANES] for k in range(nparts)]
        idxs = [jnp.full((Q_BLK, _LANES), k * _LANES, jnp.int32)
                for k in range(nparts)]
        while len(vals) > 1:
            nv, ni = [], []
            for a in range(0, len(vals), 2):
                m = vals[a + 1] > vals[a]
                nv.append(jnp.where(m, vals[a + 1], vals[a]))
                ni.append(jnp.where(m, idxs[a + 1], idxs[a]))
            vals, idxs = nv, ni
        tile_idx = idxs[0] + (lane + (j * S_BLK + g * G_BLK))
        m = vals[0] > vmax_ref[...]
        vmax_ref[...] = jnp.where(m, vals[0], vmax_ref[...])
        vidx_ref[...] = jnp.where(m, tile_idx, vidx_ref[...])

    @pl.when(j == ns_tiles - 1)
    def _finish():
        vm = vmax_ref[...]
        rowmax = jnp.max(vm, axis=1, keepdims=True)
        cand = jnp.where(vm == rowmax, vidx_ref[...], jnp.int32(2**31 - 1))
        out_ref[0, 0, :] = jnp.min(cand, axis=1)


def _gather_body(b_per_w, table_ref, idx_ref, out_ref, idx_v, rows_v, sem):
    wid = lax.axis_index("s") * _SC_CORES + lax.axis_index("c")
    base = wid * b_per_w
    pltpu.sync_copy(idx_ref.at[pl.ds(base, b_per_w)], idx_v)
    pltpu.async_copy(table_ref.at[idx_v], rows_v, sem).wait()
    pltpu.sync_copy(rows_v, out_ref.at[pl.ds(base, b_per_w)])


def kernel(x, songs, W1, b1, W2, b2, W3, b3):
    b = x.shape[0]
    n_songs, d = songs.shape
    xf = x.reshape(b, -1)

    produced = pl.pallas_call(
        _mlp_body,
        out_shape=jax.ShapeDtypeStruct((b, W3.shape[1]), jnp.float32),
    )(xf, W1, b1.reshape(1, -1), W2, b2.reshape(1, -1), W3, b3.reshape(1, -1))

    q = produced.reshape(-1, d)
    nq = q.shape[0]
    k_out = produced.shape[1] // d
    q_pad = jnp.pad(q, ((0, 0), (0, D_PAD - d)))

    ns_tiles = -(-n_songs // S_BLK)
    s_cols = ns_tiles * S_BLK
    table = jnp.pad(songs, ((0, 0), (0, D_PAD - d)))
    songs_t = table.T
    songs_t = jnp.concatenate(
        [songs_t, jnp.broadcast_to(songs_t[:, :1], (D_PAD, s_cols - n_songs))],
        axis=1)

    songs_tn = pl.pallas_call(
        _snorm_body,
        grid=(ns_tiles,),
        in_specs=[pl.BlockSpec((D_PAD, S_BLK), lambda j: (0, j))],
        out_specs=pl.BlockSpec((D_PAD, S_BLK), lambda j: (0, j)),
        out_shape=jax.ShapeDtypeStruct((D_PAD, s_cols), jnp.float32),
    )(songs_t)

    nq_tiles = nq // Q_BLK
    idx3 = pl.pallas_call(
        functools.partial(_sims_body, ns_tiles),
        grid=(nq_tiles, ns_tiles),
        in_specs=[
            pl.BlockSpec((Q_BLK, D_PAD), lambda i, j: (i, 0)),
            pl.BlockSpec((D_PAD, S_BLK), lambda i, j: (0, j)),
        ],
        out_specs=pl.BlockSpec((1, 1, Q_BLK), lambda i, j: (i, 0, 0)),
        out_shape=jax.ShapeDtypeStruct((nq_tiles, 1, Q_BLK), jnp.int32),
        scratch_shapes=[
            pltpu.VMEM((Q_BLK, D_PAD), jnp.float32),
            pltpu.VMEM((Q_BLK, _LANES), jnp.float32),
            pltpu.VMEM((Q_BLK, _LANES), jnp.int32),
        ],
        compiler_params=pltpu.CompilerParams(
            dimension_semantics=("parallel", "arbitrary"),
        ),
    )(q_pad, songs_tn)
    idx = idx3.reshape(nq)

    b_per_w = nq // _SC_WORKERS
    mesh = plsc.VectorSubcoreMesh(core_axis_name="c", subcore_axis_name="s")
    gathered = pl.kernel(
        functools.partial(_gather_body, b_per_w),
        mesh=mesh,
        out_type=jax.ShapeDtypeStruct((nq, D_PAD), jnp.float32),
        scratch_types=[
            pltpu.VMEM((b_per_w,), jnp.int32),
            pltpu.VMEM((b_per_w, D_PAD), jnp.float32),
            pltpu.SemaphoreType.DMA,
        ],
        compiler_params=pltpu.CompilerParams(use_tc_tiling_on_sc=False),
    )(table, idx)

    return gathered[:, :d].reshape(b, k_out, d)


# speedup vs baseline: 1.8485x; 1.2404x over previous
import functools

import jax
import jax.numpy as jnp
from jax import lax
from jax.experimental import pallas as pl
from jax.experimental.pallas import tpu as pltpu
from jax.experimental.pallas import tpu_sc as plsc

Q_BLK = 256
S_BLK = 8192
G_BLK = 128
D_PAD = 16
_LANES = 128

_SC_CORES = 2
_SC_SUBCORES = 16
_SC_WORKERS = _SC_CORES * _SC_SUBCORES


def _mlp_body(x_ref, w1_ref, b1_ref, w2_ref, b2_ref, w3_ref, b3_ref, out_ref):
    h = jnp.maximum(jnp.dot(x_ref[...], w1_ref[...]) + b1_ref[...], 0.0)
    h = jnp.maximum(jnp.dot(h, w2_ref[...]) + b2_ref[...], 0.0)
    out_ref[...] = jnp.dot(h, w3_ref[...]) + b3_ref[...]


def _snorm_body(s_ref, out_ref):
    st = s_ref[...]
    out_ref[...] = st / jnp.maximum(
        jnp.sqrt(jnp.sum(st * st, axis=0, keepdims=True)), 1e-8)


def _sims_body(ns_tiles, q_ref, s_ref, out_ref, qn_ref, vmax_ref, vidx_ref):
    j = pl.program_id(1)

    @pl.when(j == 0)
    def _init():
        qt = q_ref[...]
        qn_ref[...] = qt / jnp.maximum(
            jnp.sqrt(jnp.sum(qt * qt, axis=1, keepdims=True)), 1e-8)
        vmax_ref[...] = jnp.full((Q_BLK, _LANES), -jnp.inf, jnp.float32)
        vidx_ref[...] = jnp.zeros((Q_BLK, _LANES), jnp.int32)

    lane = lax.broadcasted_iota(jnp.int32, (Q_BLK, _LANES), 1)
    nparts = G_BLK // _LANES
    for g in range(S_BLK // G_BLK):
        sims = jnp.dot(qn_ref[...], s_ref[:, g * G_BLK:(g + 1) * G_BLK])
        vals = [sims[:, k * _LANES:(k + 1) * _LANES] for k in range(nparts)]
        idxs = [jnp.full((Q_BLK, _LANES), k * _LANES, jnp.int32)
                for k in range(nparts)]
        while len(vals) > 1:
            nv, ni = [], []
            for a in range(0, len(vals), 2):
                m = vals[a + 1] > vals[a]
                nv.append(jnp.where(m, vals[a + 1], vals[a]))
                ni.append(jnp.where(m, idxs[a + 1], idxs[a]))
            vals, idxs = nv, ni
        tile_idx = idxs[0] + (lane + (j * S_BLK + g * G_BLK))
        m = vals[0] > vmax_ref[...]
        vmax_ref[...] = jnp.where(m, vals[0], vmax_ref[...])
        vidx_ref[...] = jnp.where(m, tile_idx, vidx_ref[...])

    @pl.when(j == ns_tiles - 1)
    def _finish():
        vm = vmax_ref[...]
        rowmax = jnp.max(vm, axis=1, keepdims=True)
        cand = jnp.where(vm == rowmax, vidx_ref[...], jnp.int32(2**31 - 1))
        out_ref[0, 0, :] = jnp.min(cand, axis=1)


def _gather_body(b_per_w, table_ref, idx_ref, out_ref, idx_v, rows_v, sem):
    wid = lax.axis_index("s") * _SC_CORES + lax.axis_index("c")
    base = wid * b_per_w
    pltpu.sync_copy(idx_ref.at[pl.ds(base, b_per_w)], idx_v)
    pltpu.async_copy(table_ref.at[idx_v], rows_v, sem).wait()
    pltpu.sync_copy(rows_v, out_ref.at[pl.ds(base, b_per_w)])


def kernel(x, songs, W1, b1, W2, b2, W3, b3):
    b = x.shape[0]
    n_songs, d = songs.shape
    xf = x.reshape(b, -1)

    produced = pl.pallas_call(
        _mlp_body,
        out_shape=jax.ShapeDtypeStruct((b, W3.shape[1]), jnp.float32),
    )(xf, W1, b1.reshape(1, -1), W2, b2.reshape(1, -1), W3, b3.reshape(1, -1))

    q = produced.reshape(-1, d)
    nq = q.shape[0]
    k_out = produced.shape[1] // d
    q_pad = jnp.pad(q, ((0, 0), (0, D_PAD - d)))

    ns_tiles = -(-n_songs // S_BLK)
    s_cols = ns_tiles * S_BLK
    table = jnp.pad(songs, ((0, 0), (0, D_PAD - d)))
    songs_t = table.T
    songs_t = jnp.concatenate(
        [songs_t, jnp.broadcast_to(songs_t[:, :1], (D_PAD, s_cols - n_songs))],
        axis=1)

    songs_tn = pl.pallas_call(
        _snorm_body,
        grid=(ns_tiles,),
        in_specs=[pl.BlockSpec((D_PAD, S_BLK), lambda j: (0, j))],
        out_specs=pl.BlockSpec((D_PAD, S_BLK), lambda j: (0, j)),
        out_shape=jax.ShapeDtypeStruct((D_PAD, s_cols), jnp.float32),
    )(songs_t)

    nq_tiles = nq // Q_BLK
    idx3 = pl.pallas_call(
        functools.partial(_sims_body, ns_tiles),
        grid=(nq_tiles, ns_tiles),
        in_specs=[
            pl.BlockSpec((Q_BLK, D_PAD), lambda i, j: (i, 0)),
            pl.BlockSpec((D_PAD, S_BLK), lambda i, j: (0, j)),
        ],
        out_specs=pl.BlockSpec((1, 1, Q_BLK), lambda i, j: (i, 0, 0)),
        out_shape=jax.ShapeDtypeStruct((nq_tiles, 1, Q_BLK), jnp.int32),
        scratch_shapes=[
            pltpu.VMEM((Q_BLK, D_PAD), jnp.float32),
            pltpu.VMEM((Q_BLK, _LANES), jnp.float32),
            pltpu.VMEM((Q_BLK, _LANES), jnp.int32),
        ],
        compiler_params=pltpu.CompilerParams(
            dimension_semantics=("parallel", "arbitrary"),
        ),
    )(q_pad, songs_tn)
    idx = idx3.reshape(nq)

    b_per_w = nq // _SC_WORKERS
    mesh = plsc.VectorSubcoreMesh(core_axis_name="c", subcore_axis_name="s")
    gathered = pl.kernel(
        functools.partial(_gather_body, b_per_w),
        mesh=mesh,
        out_type=jax.ShapeDtypeStruct((nq, D_PAD), jnp.float32),
        scratch_types=[
            pltpu.VMEM((b_per_w,), jnp.int32),
            pltpu.VMEM((b_per_w, D_PAD), jnp.float32),
            pltpu.SemaphoreType.DMA,
        ],
        compiler_params=pltpu.CompilerParams(use_tc_tiling_on_sc=False),
    )(table, idx)

    return gathered[:, :d].reshape(b, k_out, d)


# speedup vs baseline: 2.1423x; 1.1590x over previous
import functools

import jax
import jax.numpy as jnp
from jax import lax
from jax.experimental import pallas as pl
from jax.experimental.pallas import tpu as pltpu
from jax.experimental.pallas import tpu_sc as plsc

Q_BLK = 256
S_BLK = 25088
G_BLK = 128
D_PAD = 16
_LANES = 128

_SC_CORES = 2
_SC_SUBCORES = 16
_SC_WORKERS = _SC_CORES * _SC_SUBCORES


def _mlp_body(x_ref, w1_ref, b1_ref, w2_ref, b2_ref, w3_ref, b3_ref, out_ref):
    h = jnp.maximum(jnp.dot(x_ref[...], w1_ref[...]) + b1_ref[...], 0.0)
    h = jnp.maximum(jnp.dot(h, w2_ref[...]) + b2_ref[...], 0.0)
    out_ref[...] = jnp.dot(h, w3_ref[...]) + b3_ref[...]


def _snorm_body(s_ref, out_ref):
    st = s_ref[...]
    out_ref[...] = st / jnp.maximum(
        jnp.sqrt(jnp.sum(st * st, axis=0, keepdims=True)), 1e-8)


def _sims_body(ns_tiles, q_ref, s_ref, out_ref, qn_ref, vmax_ref, vidx_ref):
    j = pl.program_id(1)

    @pl.when(j == 0)
    def _init():
        qt = q_ref[...]
        qn_ref[...] = qt / jnp.maximum(
            jnp.sqrt(jnp.sum(qt * qt, axis=1, keepdims=True)), 1e-8)
        vmax_ref[...] = jnp.full((Q_BLK, _LANES), -jnp.inf, jnp.float32)
        vidx_ref[...] = jnp.zeros((Q_BLK, _LANES), jnp.int32)

    lane = lax.broadcasted_iota(jnp.int32, (Q_BLK, _LANES), 1)
    nparts = G_BLK // _LANES
    for g in range(S_BLK // G_BLK):
        sims = jnp.dot(qn_ref[...], s_ref[:, g * G_BLK:(g + 1) * G_BLK])
        vals = [sims[:, k * _LANES:(k + 1) * _LANES] for k in range(nparts)]
        idxs = [jnp.full((Q_BLK, _LANES), k * _LANES, jnp.int32)
                for k in range(nparts)]
        while len(vals) > 1:
            nv, ni = [], []
            for a in range(0, len(vals), 2):
                m = vals[a + 1] > vals[a]
                nv.append(jnp.where(m, vals[a + 1], vals[a]))
                ni.append(jnp.where(m, idxs[a + 1], idxs[a]))
            vals, idxs = nv, ni
        tile_idx = idxs[0] + (lane + (j * S_BLK + g * G_BLK))
        m = vals[0] > vmax_ref[...]
        vmax_ref[...] = jnp.where(m, vals[0], vmax_ref[...])
        vidx_ref[...] = jnp.where(m, tile_idx, vidx_ref[...])

    @pl.when(j == ns_tiles - 1)
    def _finish():
        vm = vmax_ref[...]
        rowmax = jnp.max(vm, axis=1, keepdims=True)
        cand = jnp.where(vm == rowmax, vidx_ref[...], jnp.int32(2**31 - 1))
        out_ref[0, 0, :] = jnp.min(cand, axis=1)


def _gather_body(b_per_w, table_ref, idx_ref, out_ref, idx_v, rows_v, sem):
    wid = lax.axis_index("s") * _SC_CORES + lax.axis_index("c")
    base = wid * b_per_w
    pltpu.sync_copy(idx_ref.at[pl.ds(base, b_per_w)], idx_v)
    pltpu.async_copy(table_ref.at[idx_v], rows_v, sem).wait()
    pltpu.sync_copy(rows_v, out_ref.at[pl.ds(base, b_per_w)])


def kernel(x, songs, W1, b1, W2, b2, W3, b3):
    b = x.shape[0]
    n_songs, d = songs.shape
    xf = x.reshape(b, -1)

    produced = pl.pallas_call(
        _mlp_body,
        out_shape=jax.ShapeDtypeStruct((b, W3.shape[1]), jnp.float32),
    )(xf, W1, b1.reshape(1, -1), W2, b2.reshape(1, -1), W3, b3.reshape(1, -1))

    q = produced.reshape(-1, d)
    nq = q.shape[0]
    k_out = produced.shape[1] // d
    q_pad = jnp.pad(q, ((0, 0), (0, D_PAD - d)))

    ns_tiles = -(-n_songs // S_BLK)
    s_cols = ns_tiles * S_BLK
    table = jnp.pad(songs, ((0, 0), (0, D_PAD - d)))
    songs_t = table.T
    songs_t = jnp.concatenate(
        [songs_t, jnp.broadcast_to(songs_t[:, :1], (D_PAD, s_cols - n_songs))],
        axis=1)

    songs_tn = pl.pallas_call(
        _snorm_body,
        grid=(ns_tiles,),
        in_specs=[pl.BlockSpec((D_PAD, S_BLK), lambda j: (0, j))],
        out_specs=pl.BlockSpec((D_PAD, S_BLK), lambda j: (0, j)),
        out_shape=jax.ShapeDtypeStruct((D_PAD, s_cols), jnp.float32),
    )(songs_t)

    nq_tiles = nq // Q_BLK
    idx3 = pl.pallas_call(
        functools.partial(_sims_body, ns_tiles),
        grid=(nq_tiles, ns_tiles),
        in_specs=[
            pl.BlockSpec((Q_BLK, D_PAD), lambda i, j: (i, 0)),
            pl.BlockSpec((D_PAD, S_BLK), lambda i, j: (0, j)),
        ],
        out_specs=pl.BlockSpec((1, 1, Q_BLK), lambda i, j: (i, 0, 0)),
        out_shape=jax.ShapeDtypeStruct((nq_tiles, 1, Q_BLK), jnp.int32),
        scratch_shapes=[
            pltpu.VMEM((Q_BLK, D_PAD), jnp.float32),
            pltpu.VMEM((Q_BLK, _LANES), jnp.float32),
            pltpu.VMEM((Q_BLK, _LANES), jnp.int32),
        ],
        compiler_params=pltpu.CompilerParams(
            dimension_semantics=("parallel", "arbitrary"),
        ),
    )(q_pad, songs_tn)
    idx = idx3.reshape(nq)

    b_per_w = nq // _SC_WORKERS
    mesh = plsc.VectorSubcoreMesh(core_axis_name="c", subcore_axis_name="s")
    gathered = pl.kernel(
        functools.partial(_gather_body, b_per_w),
        mesh=mesh,
        out_type=jax.ShapeDtypeStruct((nq, D_PAD), jnp.float32),
        scratch_types=[
            pltpu.VMEM((b_per_w,), jnp.int32),
            pltpu.VMEM((b_per_w, D_PAD), jnp.float32),
            pltpu.SemaphoreType.DMA,
        ],
        compiler_params=pltpu.CompilerParams(use_tc_tiling_on_sc=False),
    )(table, idx)

    return gathered[:, :d].reshape(b, k_out, d)


# speedup vs baseline: 2.1975x; 1.0258x over previous
import functools

import jax
import jax.numpy as jnp
from jax import lax
from jax.experimental import pallas as pl
from jax.experimental.pallas import tpu as pltpu
from jax.experimental.pallas import tpu_sc as plsc

Q_BLK = 256
S_BLK = 50176
G_BLK = 128
D_PAD = 16
_LANES = 128

_SC_CORES = 2
_SC_SUBCORES = 16
_SC_WORKERS = _SC_CORES * _SC_SUBCORES


def _mlp_body(x_ref, w1_ref, b1_ref, w2_ref, b2_ref, w3_ref, b3_ref, out_ref):
    h = jnp.maximum(jnp.dot(x_ref[...], w1_ref[...]) + b1_ref[...], 0.0)
    h = jnp.maximum(jnp.dot(h, w2_ref[...]) + b2_ref[...], 0.0)
    out_ref[...] = jnp.dot(h, w3_ref[...]) + b3_ref[...]


def _snorm_body(s_ref, out_ref):
    st = s_ref[...]
    out_ref[...] = st / jnp.maximum(
        jnp.sqrt(jnp.sum(st * st, axis=0, keepdims=True)), 1e-8)


def _sims_body(ns_tiles, q_ref, s_ref, out_ref, qn_ref, vmax_ref, vidx_ref):
    j = pl.program_id(1)

    @pl.when(j == 0)
    def _init():
        qt = q_ref[...]
        qn_ref[...] = qt / jnp.maximum(
            jnp.sqrt(jnp.sum(qt * qt, axis=1, keepdims=True)), 1e-8)
        vmax_ref[...] = jnp.full((Q_BLK, _LANES), -jnp.inf, jnp.float32)
        vidx_ref[...] = jnp.zeros((Q_BLK, _LANES), jnp.int32)

    lane = lax.broadcasted_iota(jnp.int32, (Q_BLK, _LANES), 1)
    nparts = G_BLK // _LANES
    for g in range(S_BLK // G_BLK):
        sims = jnp.dot(qn_ref[...], s_ref[:, g * G_BLK:(g + 1) * G_BLK])
        vals = [sims[:, k * _LANES:(k + 1) * _LANES] for k in range(nparts)]
        idxs = [jnp.full((Q_BLK, _LANES), k * _LANES, jnp.int32)
                for k in range(nparts)]
        while len(vals) > 1:
            nv, ni = [], []
            for a in range(0, len(vals), 2):
                m = vals[a + 1] > vals[a]
                nv.append(jnp.where(m, vals[a + 1], vals[a]))
                ni.append(jnp.where(m, idxs[a + 1], idxs[a]))
            vals, idxs = nv, ni
        tile_idx = idxs[0] + (lane + (j * S_BLK + g * G_BLK))
        m = vals[0] > vmax_ref[...]
        vmax_ref[...] = jnp.where(m, vals[0], vmax_ref[...])
        vidx_ref[...] = jnp.where(m, tile_idx, vidx_ref[...])

    @pl.when(j == ns_tiles - 1)
    def _finish():
        vm = vmax_ref[...]
        rowmax = jnp.max(vm, axis=1, keepdims=True)
        cand = jnp.where(vm == rowmax, vidx_ref[...], jnp.int32(2**31 - 1))
        out_ref[0, 0, :] = jnp.min(cand, axis=1)


def _gather_body(b_per_w, table_ref, idx_ref, out_ref, idx_v, rows_v, sem):
    wid = lax.axis_index("s") * _SC_CORES + lax.axis_index("c")
    base = wid * b_per_w
    pltpu.sync_copy(idx_ref.at[pl.ds(base, b_per_w)], idx_v)
    pltpu.async_copy(table_ref.at[idx_v], rows_v, sem).wait()
    pltpu.sync_copy(rows_v, out_ref.at[pl.ds(base, b_per_w)])


def kernel(x, songs, W1, b1, W2, b2, W3, b3):
    b = x.shape[0]
    n_songs, d = songs.shape
    xf = x.reshape(b, -1)

    produced = pl.pallas_call(
        _mlp_body,
        out_shape=jax.ShapeDtypeStruct((b, W3.shape[1]), jnp.float32),
    )(xf, W1, b1.reshape(1, -1), W2, b2.reshape(1, -1), W3, b3.reshape(1, -1))

    q = produced.reshape(-1, d)
    nq = q.shape[0]
    k_out = produced.shape[1] // d
    q_pad = jnp.pad(q, ((0, 0), (0, D_PAD - d)))

    ns_tiles = -(-n_songs // S_BLK)
    s_cols = ns_tiles * S_BLK
    table = jnp.pad(songs, ((0, 0), (0, D_PAD - d)))
    songs_t = table.T
    songs_t = jnp.concatenate(
        [songs_t, jnp.broadcast_to(songs_t[:, :1], (D_PAD, s_cols - n_songs))],
        axis=1)

    songs_tn = pl.pallas_call(
        _snorm_body,
        grid=(ns_tiles,),
        in_specs=[pl.BlockSpec((D_PAD, S_BLK), lambda j: (0, j))],
        out_specs=pl.BlockSpec((D_PAD, S_BLK), lambda j: (0, j)),
        out_shape=jax.ShapeDtypeStruct((D_PAD, s_cols), jnp.float32),
    )(songs_t)

    nq_tiles = nq // Q_BLK
    idx3 = pl.pallas_call(
        functools.partial(_sims_body, ns_tiles),
        grid=(nq_tiles, ns_tiles),
        in_specs=[
            pl.BlockSpec((Q_BLK, D_PAD), lambda i, j: (i, 0)),
            pl.BlockSpec((D_PAD, S_BLK), lambda i, j: (0, j)),
        ],
        out_specs=pl.BlockSpec((1, 1, Q_BLK), lambda i, j: (i, 0, 0)),
        out_shape=jax.ShapeDtypeStruct((nq_tiles, 1, Q_BLK), jnp.int32),
        scratch_shapes=[
            pltpu.VMEM((Q_BLK, D_PAD), jnp.float32),
            pltpu.VMEM((Q_BLK, _LANES), jnp.float32),
            pltpu.VMEM((Q_BLK, _LANES), jnp.int32),
        ],
        compiler_params=pltpu.CompilerParams(
            dimension_semantics=("parallel", "arbitrary"),
        ),
    )(q_pad, songs_tn)
    idx = idx3.reshape(nq)

    b_per_w = nq // _SC_WORKERS
    mesh = plsc.VectorSubcoreMesh(core_axis_name="c", subcore_axis_name="s")
    gathered = pl.kernel(
        functools.partial(_gather_body, b_per_w),
        mesh=mesh,
        out_type=jax.ShapeDtypeStruct((nq, D_PAD), jnp.float32),
        scratch_types=[
            pltpu.VMEM((b_per_w,), jnp.int32),
            pltpu.VMEM((b_per_w, D_PAD), jnp.float32),
            pltpu.SemaphoreType.DMA,
        ],
        compiler_params=pltpu.CompilerParams(use_tc_tiling_on_sc=False),
    )(table, idx)

    return gathered[:, :d].reshape(b, k_out, d)
